# baseline (speedup 1.0000x reference)
import functools

import jax
import jax.numpy as jnp
from jax import lax
from jax.experimental import pallas as pl
from jax.experimental.pallas import tpu as pltpu
from jax.experimental.pallas import tpu_sc as plsc

N = 10000
E = 320000
D = 128
B = 16
NP = 10240
NP2 = 2 * NP
NSC = 2
NTL = 16
CH = 128
ECH = 2560
TCH = ECH // NTL
RPT = NP // NTL
BM = 1024
NBLK = NP2 // BM


def _mesh():
    return plsc.VectorSubcoreMesh(
        core_axis_name="c", subcore_axis_name="s", num_cores=NSC, num_subcores=NTL
    )


def _sc_deg_body(dst1, dst2, zbig, ones_h, out, idx_v, ones_v, acc, sem):
    c = lax.axis_index("c")
    s = lax.axis_index("s")
    pltpu.sync_copy(ones_h, ones_v)
    pltpu.sync_copy(zbig.at[pl.ds(s * RPT, RPT)], acc.at[pl.ds(s * RPT, RPT)])

    @pl.when(c == 0)
    def _():
        pltpu.sync_copy(dst1.at[pl.ds(s * TCH, TCH)], idx_v)

    @pl.when(c == 1)
    def _():
        pltpu.sync_copy(dst2.at[pl.ds(s * TCH, TCH)], idx_v)

    plsc.subcore_barrier()

    def body(j, carry):
        pltpu.sync_copy(ones_v, acc.at[idx_v.at[j]], add=True)
        return carry

    lax.fori_loop(0, TCH, body, 0)
    plsc.subcore_barrier()
    pltpu.sync_copy(acc.at[pl.ds(s * RPT, RPT)], out.at[pl.ds(c * NP + s * RPT, RPT)])


def _sc_deg(dst1, dst2, zbig, ones_h):
    return pl.kernel(
        _sc_deg_body,
        out_type=jax.ShapeDtypeStruct((NP2, D), jnp.float32),
        mesh=_mesh(),
        scratch_types=[
            pltpu.VMEM((TCH, CH), jnp.int32),
            pltpu.VMEM((CH, D), jnp.float32),
            pltpu.VMEM_SHARED((NP, D), jnp.float32),
            pltpu.SemaphoreType.DMA,
        ],
    )(dst1, dst2, zbig, ones_h)


STG = 40
NSTG = TCH // STG


def _sc_agg_body(ycat, src1, dst1, src2, dst2, zbig, out,
                 src_v, dst_v, rows, acc, sem):
    c = lax.axis_index("c")
    s = lax.axis_index("s")
    pltpu.sync_copy(zbig.at[pl.ds(s * RPT, RPT)], acc.at[pl.ds(s * RPT, RPT)])
    plsc.subcore_barrier()

    def run(src_hbm, dst_hbm):
        def stage(st, carry):
            pltpu.sync_copy(src_hbm.at[pl.ds(s * TCH + st * STG, STG)], src_v)
            pltpu.sync_copy(dst_hbm.at[pl.ds(s * TCH + st * STG, STG)], dst_v)

            def body(j, carry2):
                pltpu.async_copy(ycat.at[src_v.at[j]], rows, sem).wait()
                pltpu.sync_copy(rows, acc.at[dst_v.at[j]], add=True)
                return carry2

            return lax.fori_loop(0, STG, body, carry)

        lax.fori_loop(0, NSTG, stage, 0)

    @pl.when(c == 0)
    def _():
        run(src1, dst1)

    @pl.when(c == 1)
    def _():
        run(src2, dst2)

    plsc.subcore_barrier()
    pltpu.sync_copy(acc.at[pl.ds(s * RPT, RPT)], out.at[pl.ds(c * NP + s * RPT, RPT)])


def _sc_agg(ycat, src1, dst1, src2, dst2, zbig):
    return pl.kernel(
        _sc_agg_body,
        out_type=jax.ShapeDtypeStruct((NP2, D), jnp.float32),
        mesh=_mesh(),
        scratch_types=[
            pltpu.VMEM((STG, CH), jnp.int32),
            pltpu.VMEM((STG, CH), jnp.int32),
            pltpu.VMEM((CH, D), jnp.float32),
            pltpu.VMEM_SHARED((NP, D), jnp.float32),
            pltpu.SemaphoreType.DMA,
        ],
    )(ycat, src1, dst1, src2, dst2, zbig)


def _tc_prep_body(x_ref, deg_ref, w0_ref, b0_ref, wc_ref, out_ref, y_ref):
    x = x_ref[0]
    o = jnp.maximum(
        lax.dot_general(x, w0_ref[0], (((1,), (1,)), ((), ())),
                        preferred_element_type=jnp.float32) + b0_ref[0], 0.0)
    dinv = lax.rsqrt(deg_ref[0][:, 0:1] + 1.0)
    out_ref[0] = o
    y_ref[0] = dinv * jnp.dot(o, wc_ref[0], preferred_element_type=jnp.float32)


def _tc_prep(xcat, degcat, w0, b0, wc):
    f32 = jnp.float32
    return pl.pallas_call(
        _tc_prep_body,
        grid=(NBLK,),
        in_specs=[
            pl.BlockSpec((1, BM, D), lambda i: (0, i, 0)),
            pl.BlockSpec((1, BM, 16), lambda i: (0, i, 0)),
            pl.BlockSpec((1, D, D), lambda i: (i // (NBLK // 2), 0, 0)),
            pl.BlockSpec((1, 1, D), lambda i: (i // (NBLK // 2), 0, 0)),
            pl.BlockSpec((1, D, D), lambda i: (i // (NBLK // 2), 0, 0)),
        ],
        out_specs=[
            pl.BlockSpec((1, BM, D), lambda i: (0, i, 0)),
            pl.BlockSpec((1, BM, D), lambda i: (0, i, 0)),
        ],
        out_shape=[
            jax.ShapeDtypeStruct((1, NP2, D), f32),
            jax.ShapeDtypeStruct((1, NP2, D), f32),
        ],
    )(xcat[None], degcat[None], w0, b0, wc)


def _tc_layer_body(acc_ref, y_ref, h_ref, deg_ref, wc_ref, wih_ref, whh_ref,
                   bih_ref, bhh_ref, bc_ref, hn_ref, yn_ref):
    dinv = lax.rsqrt(deg_ref[0][:, 0:1] + 1.0)
    m = jnp.maximum(dinv * (acc_ref[0] + y_ref[0]) + bc_ref[0], 0.0)
    h = h_ref[0]
    gi = lax.dot_general(m, wih_ref[0], (((1,), (1,)), ((), ())),
                         preferred_element_type=jnp.float32) + bih_ref[0]
    gh = lax.dot_general(h, whh_ref[0], (((1,), (1,)), ((), ())),
                         preferred_element_type=jnp.float32) + bhh_ref[0]
    r = jax.nn.sigmoid(gi[:, 0:D] + gh[:, 0:D])
    z = jax.nn.sigmoid(gi[:, D:2 * D] + gh[:, D:2 * D])
    nn_ = jnp.tanh(gi[:, 2 * D:3 * D] + r * gh[:, 2 * D:3 * D])
    hn = (1.0 - z) * nn_ + z * h
    hn_ref[0] = hn
    yn_ref[0] = dinv * jnp.dot(hn, wc_ref[0], preferred_element_type=jnp.float32)


def _tc_layer(acccat, ycat, hcat, degcat, wc, wih, whh, bih, bhh, bc):
    f32 = jnp.float32
    wspec = lambda r, c: pl.BlockSpec((1, r, c), lambda i: (i // (NBLK // 2), 0, 0))
    return pl.pallas_call(
        _tc_layer_body,
        grid=(NBLK,),
        in_specs=[
            pl.BlockSpec((1, BM, D), lambda i: (0, i, 0)),
            pl.BlockSpec((1, BM, D), lambda i: (0, i, 0)),
            pl.BlockSpec((1, BM, D), lambda i: (0, i, 0)),
            pl.BlockSpec((1, BM, 16), lambda i: (0, i, 0)),
            wspec(D, D),
            wspec(3 * D, D),
            wspec(3 * D, D),
            wspec(1, 3 * D),
            wspec(1, 3 * D),
            wspec(1, D),
        ],
        out_specs=[
            pl.BlockSpec((1, BM, D), lambda i: (0, i, 0)),
            pl.BlockSpec((1, BM, D), lambda i: (0, i, 0)),
        ],
        out_shape=[
            jax.ShapeDtypeStruct((1, NP2, D), f32),
            jax.ShapeDtypeStruct((1, NP2, D), f32),
        ],
    )(acccat[None], ycat[None], hcat[None], degcat[None], wc, wih, whh, bih, bhh, bc)


def _set2set_in_kernel(x, bcol16, brow16, wih, whh, bih, bhh):
    mtb = bcol16 == lax.broadcasted_iota(jnp.int32, (NP, 16), 1)
    mtf = mtb.astype(jnp.float32)
    mf = (brow16 == lax.broadcasted_iota(jnp.int32, (B, NP), 0)).astype(jnp.float32)
    h = jnp.zeros((B, D), jnp.float32)
    cc = jnp.zeros((B, D), jnp.float32)
    q_star = jnp.zeros((B, 2 * D), jnp.float32)
    for _ in range(3):
        gates = (lax.dot_general(q_star, wih, (((1,), (1,)), ((), ())),
                                 preferred_element_type=jnp.float32) + bih
                 + lax.dot_general(h, whh, (((1,), (1,)), ((), ())),
                                   preferred_element_type=jnp.float32) + bhh)
        ii = jax.nn.sigmoid(gates[:, 0:D])
        ff = jax.nn.sigmoid(gates[:, D:2 * D])
        gg = jnp.tanh(gates[:, 2 * D:3 * D])
        oo = jax.nn.sigmoid(gates[:, 3 * D:4 * D])
        cc = ff * cc + ii * gg
        h = oo * jnp.tanh(cc)
        q = h
        ee = lax.dot_general(x, q, (((1,), (1,)), ((), ())),
                             preferred_element_type=jnp.float32)
        e = jnp.sum(ee * mtf, axis=1, keepdims=True)
        emax = jnp.max(jnp.where(mtb, jnp.broadcast_to(e, (NP, 16)), -jnp.inf),
                       axis=0, keepdims=True)
        emax = jnp.where(emax > -1e30, emax, 0.0)
        emax_sel = jnp.sum(emax * mtf, axis=1, keepdims=True)
        ex = jnp.exp(e - emax_sel)
        denom = jnp.sum(mtf * jnp.broadcast_to(ex, (NP, 16)), axis=0, keepdims=True)
        denom_sel = jnp.sum(denom * mtf, axis=1, keepdims=True)
        a = ex / (denom_sel + 1e-16)
        r = jnp.dot(mf, a * x, preferred_element_type=jnp.float32)
        q_star = jnp.concatenate([q, r], axis=1)
    return q_star


def _tc_final_body(h_ref, b1c_ref, b1r_ref, b2c_ref, b2r_ref,
                   wih1_ref, whh1_ref, bih1_ref, bhh1_ref,
                   wih2_ref, whh2_ref, bih2_ref, bhh2_ref,
                   w1_ref, b1_ref, w2_ref, b2_ref, o_ref):
    g1 = _set2set_in_kernel(h_ref[0:NP], b1c_ref[...], b1r_ref[...],
                            wih1_ref[...], whh1_ref[...], bih1_ref[...], bhh1_ref[...])
    g2 = _set2set_in_kernel(h_ref[NP:NP2], b2c_ref[...], b2r_ref[...],
                            wih2_ref[...], whh2_ref[...], bih2_ref[...], bhh2_ref[...])
    cat = jnp.concatenate([g1, g2], axis=1)
    hfc = jnp.maximum(
        lax.dot_general(cat, w1_ref[...], (((1,), (1,)), ((), ())),
                        preferred_element_type=jnp.float32) + b1_ref[...], 0.0)
    o_ref[...] = jnp.sum(hfc * w2_ref[...], axis=1, keepdims=True) + b2_ref[...]


def _tc_final(hcat, b1c, b1r, b2c, b2r, p1, p2, fc1_W, fc1_b, fc2_W, fc2_b):
    return pl.pallas_call(
        _tc_final_body,
        out_shape=jax.ShapeDtypeStruct((B, 1), jnp.float32),
    )(hcat, b1c, b1r, b2c, b2r,
      p1[0], p1[1], p1[2], p1[3],
      p2[0], p2[1], p2[2], p2[3],
      fc1_W, fc1_b[None, :], fc2_W, fc2_b[None, :])


def kernel(x1, x2, edge_index1, edge_index2, x1_batch, x2_batch,
           e1_lin0_W, e1_lin0_b, e1_conv_W, e1_conv_b,
           e1_gru_Wih, e1_gru_Whh, e1_gru_bih, e1_gru_bhh,
           e1_lstm_Wih, e1_lstm_Whh, e1_lstm_bih, e1_lstm_bhh,
           e2_lin0_W, e2_lin0_b, e2_conv_W, e2_conv_b,
           e2_gru_Wih, e2_gru_Whh, e2_gru_bih, e2_gru_bhh,
           e2_lstm_Wih, e2_lstm_Whh, e2_lstm_bih, e2_lstm_bhh,
           fc1_W, fc1_b, fc2_W, fc2_b):
    f32 = jnp.float32
    epad = ECH * CH - E

    def edges2d(v, off):
        return jnp.pad(v, (0, epad), constant_values=N).reshape(ECH, CH) + off

    src1 = edges2d(edge_index1[0], 0)
    dst1 = edges2d(edge_index1[1], 0)
    src2 = edges2d(edge_index2[0], NP)
    dst2 = edges2d(edge_index2[1], 0)
    zbig = jnp.zeros((NP, D), f32)
    ones_h = jnp.ones((CH, D), f32)

    degcat = _sc_deg(dst1, dst2, zbig, ones_h)[:, 0:16]

    xcat = jnp.concatenate([
        jnp.pad(x1, ((0, NP - N), (0, 0))),
        jnp.pad(x2, ((0, NP - N), (0, 0))),
    ], axis=0)
    w0 = jnp.stack([e1_lin0_W, e2_lin0_W])
    b0 = jnp.stack([e1_lin0_b[None, :], e2_lin0_b[None, :]])
    wc = jnp.stack([e1_conv_W, e2_conv_W])
    bc = jnp.stack([e1_conv_b[None, :], e2_conv_b[None, :]])
    wih = jnp.stack([e1_gru_Wih, e2_gru_Wih])
    whh = jnp.stack([e1_gru_Whh, e2_gru_Whh])
    bih = jnp.stack([e1_gru_bih[None, :], e2_gru_bih[None, :]])
    bhh = jnp.stack([e1_gru_bhh[None, :], e2_gru_bhh[None, :]])

    hcat, ycat = _tc_prep(xcat, degcat, w0, b0, wc)
    hcat, ycat = hcat[0], ycat[0]
    for _ in range(3):
        acccat = _sc_agg(ycat, src1, dst1, src2, dst2, zbig)
        hcat, ycat = _tc_layer(acccat, ycat, hcat, degcat, wc, wih, whh, bih, bhh, bc)
        hcat, ycat = hcat[0], ycat[0]

    bpad = NP - N
    b1p = jnp.pad(x1_batch, (0, bpad), constant_values=B)
    b2p = jnp.pad(x2_batch, (0, bpad), constant_values=B)
    b1c = jnp.broadcast_to(b1p[:, None], (NP, 16))
    b2c = jnp.broadcast_to(b2p[:, None], (NP, 16))
    b1r = jnp.broadcast_to(b1p[None, :], (B, NP))
    b2r = jnp.broadcast_to(b2p[None, :], (B, NP))
    p1 = (e1_lstm_Wih, e1_lstm_Whh, e1_lstm_bih[None, :], e1_lstm_bhh[None, :])
    p2 = (e2_lstm_Wih, e2_lstm_Whh, e2_lstm_bih[None, :], e2_lstm_bhh[None, :])
    out = _tc_final(hcat, b1c, b1r, b2c, b2r, p1, p2, fc1_W, fc1_b, fc2_W, fc2_b)
    return out.reshape(-1)


# speedup vs baseline: 1.1392x; 1.1392x over previous
import functools

import jax
import jax.numpy as jnp
from jax import lax
from jax.experimental import pallas as pl
from jax.experimental.pallas import tpu as pltpu
from jax.experimental.pallas import tpu_sc as plsc

N = 10000
E = 320000
D = 128
B = 16
NP = 10240
NP2 = 2 * NP
NSC = 2
NTL = 16
CH = 128
ECH = 2560
TCH = ECH // NTL
RPT = NP // NTL
BM = 1024
NBLK = NP2 // BM


def _mesh():
    return plsc.VectorSubcoreMesh(
        core_axis_name="c", subcore_axis_name="s", num_cores=NSC, num_subcores=NTL
    )


def _sc_deg_body(dst1, dst2, zbig, ones_h, out, idx_v, ones_v, acc, sem):
    c = lax.axis_index("c")
    s = lax.axis_index("s")
    pltpu.sync_copy(ones_h, ones_v)
    pltpu.sync_copy(zbig.at[pl.ds(s * RPT, RPT)], acc.at[pl.ds(s * RPT, RPT)])

    @pl.when(c == 0)
    def _():
        pltpu.sync_copy(dst1.at[pl.ds(s * TCH, TCH)], idx_v)

    @pl.when(c == 1)
    def _():
        pltpu.sync_copy(dst2.at[pl.ds(s * TCH, TCH)], idx_v)

    plsc.subcore_barrier()

    def body(j, carry):
        pltpu.sync_copy(ones_v, acc.at[idx_v.at[j]], add=True)
        return carry

    lax.fori_loop(0, TCH, body, 0)
    plsc.subcore_barrier()
    pltpu.sync_copy(acc.at[pl.ds(s * RPT, RPT)], out.at[pl.ds(c * NP + s * RPT, RPT)])


def _sc_deg(dst1, dst2, zbig, ones_h):
    return pl.kernel(
        _sc_deg_body,
        out_type=jax.ShapeDtypeStruct((NP2, D), jnp.float32),
        mesh=_mesh(),
        scratch_types=[
            pltpu.VMEM((TCH, CH), jnp.int32),
            pltpu.VMEM((CH, D), jnp.float32),
            pltpu.VMEM_SHARED((NP, D), jnp.float32),
            pltpu.SemaphoreType.DMA,
        ],
    )(dst1, dst2, zbig, ones_h)


STG = 40
NSTG = TCH // STG
UNR = 8


def _sc_agg_body(ycat, src1, dst1, src2, dst2, zbig, out,
                 src_v, dst_v, rows, rows1, acc, sem, sem1):
    c = lax.axis_index("c")
    s = lax.axis_index("s")
    pltpu.sync_copy(zbig.at[pl.ds(s * RPT, RPT)], acc.at[pl.ds(s * RPT, RPT)])
    plsc.subcore_barrier()

    def run(src_hbm, dst_hbm):
        bufs = (rows, rows1)
        sems = (sem, sem1)

        def stage(st, carry):
            pltpu.sync_copy(src_hbm.at[pl.ds(s * TCH + st * STG, STG)], src_v)
            pltpu.sync_copy(dst_hbm.at[pl.ds(s * TCH + st * STG, STG)], dst_v)

            def block(ko, carry2):
                base = ko * UNR
                ds = [
                    pltpu.async_copy(ycat.at[src_v.at[base]], bufs[0], sems[0]),
                    pltpu.async_copy(ycat.at[src_v.at[base + 1]], bufs[1], sems[1]),
                ]
                for k in range(UNR):
                    ds[k].wait()
                    pltpu.sync_copy(bufs[k % 2], acc.at[dst_v.at[base + k]], add=True)
                    if k + 2 < UNR:
                        ds.append(pltpu.async_copy(
                            ycat.at[src_v.at[base + k + 2]], bufs[k % 2], sems[k % 2]))
                return carry2

            return lax.fori_loop(0, STG // UNR, block, carry)

        lax.fori_loop(0, NSTG, stage, 0)

    @pl.when(c == 0)
    def _():
        run(src1, dst1)

    @pl.when(c == 1)
    def _():
        run(src2, dst2)

    plsc.subcore_barrier()
    pltpu.sync_copy(acc.at[pl.ds(s * RPT, RPT)], out.at[pl.ds(c * NP + s * RPT, RPT)])


def _sc_agg(ycat, src1, dst1, src2, dst2, zbig):
    return pl.kernel(
        _sc_agg_body,
        out_type=jax.ShapeDtypeStruct((NP2, D), jnp.float32),
        mesh=_mesh(),
        scratch_types=[
            pltpu.VMEM((STG, CH), jnp.int32),
            pltpu.VMEM((STG, CH), jnp.int32),
            pltpu.VMEM((CH, D), jnp.float32),
            pltpu.VMEM((CH, D), jnp.float32),
            pltpu.VMEM_SHARED((NP, D), jnp.float32),
            pltpu.SemaphoreType.DMA,
            pltpu.SemaphoreType.DMA,
        ],
    )(ycat, src1, dst1, src2, dst2, zbig)


def _tc_prep_body(x_ref, deg_ref, w0_ref, b0_ref, wc_ref, out_ref, y_ref):
    x = x_ref[0]
    o = jnp.maximum(
        lax.dot_general(x, w0_ref[0], (((1,), (1,)), ((), ())),
                        preferred_element_type=jnp.float32) + b0_ref[0], 0.0)
    dinv = lax.rsqrt(deg_ref[0][:, 0:1] + 1.0)
    out_ref[0] = o
    y_ref[0] = dinv * jnp.dot(o, wc_ref[0], preferred_element_type=jnp.float32)


def _tc_prep(xcat, degcat, w0, b0, wc):
    f32 = jnp.float32
    return pl.pallas_call(
        _tc_prep_body,
        grid=(NBLK,),
        in_specs=[
            pl.BlockSpec((1, BM, D), lambda i: (0, i, 0)),
            pl.BlockSpec((1, BM, 16), lambda i: (0, i, 0)),
            pl.BlockSpec((1, D, D), lambda i: (i // (NBLK // 2), 0, 0)),
            pl.BlockSpec((1, 1, D), lambda i: (i // (NBLK // 2), 0, 0)),
            pl.BlockSpec((1, D, D), lambda i: (i // (NBLK // 2), 0, 0)),
        ],
        out_specs=[
            pl.BlockSpec((1, BM, D), lambda i: (0, i, 0)),
            pl.BlockSpec((1, BM, D), lambda i: (0, i, 0)),
        ],
        out_shape=[
            jax.ShapeDtypeStruct((1, NP2, D), f32),
            jax.ShapeDtypeStruct((1, NP2, D), f32),
        ],
    )(xcat[None], degcat[None], w0, b0, wc)


def _tc_layer_body(acc_ref, y_ref, h_ref, deg_ref, wc_ref, wih_ref, whh_ref,
                   bih_ref, bhh_ref, bc_ref, hn_ref, yn_ref):
    dinv = lax.rsqrt(deg_ref[0][:, 0:1] + 1.0)
    m = jnp.maximum(dinv * (acc_ref[0] + y_ref[0]) + bc_ref[0], 0.0)
    h = h_ref[0]
    gi = lax.dot_general(m, wih_ref[0], (((1,), (1,)), ((), ())),
                         preferred_element_type=jnp.float32) + bih_ref[0]
    gh = lax.dot_general(h, whh_ref[0], (((1,), (1,)), ((), ())),
                         preferred_element_type=jnp.float32) + bhh_ref[0]
    r = jax.nn.sigmoid(gi[:, 0:D] + gh[:, 0:D])
    z = jax.nn.sigmoid(gi[:, D:2 * D] + gh[:, D:2 * D])
    nn_ = jnp.tanh(gi[:, 2 * D:3 * D] + r * gh[:, 2 * D:3 * D])
    hn = (1.0 - z) * nn_ + z * h
    hn_ref[0] = hn
    yn_ref[0] = dinv * jnp.dot(hn, wc_ref[0], preferred_element_type=jnp.float32)


def _tc_layer(acccat, ycat, hcat, degcat, wc, wih, whh, bih, bhh, bc):
    f32 = jnp.float32
    wspec = lambda r, c: pl.BlockSpec((1, r, c), lambda i: (i // (NBLK // 2), 0, 0))
    return pl.pallas_call(
        _tc_layer_body,
        grid=(NBLK,),
        in_specs=[
            pl.BlockSpec((1, BM, D), lambda i: (0, i, 0)),
            pl.BlockSpec((1, BM, D), lambda i: (0, i, 0)),
            pl.BlockSpec((1, BM, D), lambda i: (0, i, 0)),
            pl.BlockSpec((1, BM, 16), lambda i: (0, i, 0)),
            wspec(D, D),
            wspec(3 * D, D),
            wspec(3 * D, D),
            wspec(1, 3 * D),
            wspec(1, 3 * D),
            wspec(1, D),
        ],
        out_specs=[
            pl.BlockSpec((1, BM, D), lambda i: (0, i, 0)),
            pl.BlockSpec((1, BM, D), lambda i: (0, i, 0)),
        ],
        out_shape=[
            jax.ShapeDtypeStruct((1, NP2, D), f32),
            jax.ShapeDtypeStruct((1, NP2, D), f32),
        ],
    )(acccat[None], ycat[None], hcat[None], degcat[None], wc, wih, whh, bih, bhh, bc)


def _set2set_in_kernel(x, bcol16, brow16, wih, whh, bih, bhh):
    mtb = bcol16 == lax.broadcasted_iota(jnp.int32, (NP, 16), 1)
    mtf = mtb.astype(jnp.float32)
    mf = (brow16 == lax.broadcasted_iota(jnp.int32, (B, NP), 0)).astype(jnp.float32)
    h = jnp.zeros((B, D), jnp.float32)
    cc = jnp.zeros((B, D), jnp.float32)
    q_star = jnp.zeros((B, 2 * D), jnp.float32)
    for _ in range(3):
        gates = (lax.dot_general(q_star, wih, (((1,), (1,)), ((), ())),
                                 preferred_element_type=jnp.float32) + bih
                 + lax.dot_general(h, whh, (((1,), (1,)), ((), ())),
                                   preferred_element_type=jnp.float32) + bhh)
        ii = jax.nn.sigmoid(gates[:, 0:D])
        ff = jax.nn.sigmoid(gates[:, D:2 * D])
        gg = jnp.tanh(gates[:, 2 * D:3 * D])
        oo = jax.nn.sigmoid(gates[:, 3 * D:4 * D])
        cc = ff * cc + ii * gg
        h = oo * jnp.tanh(cc)
        q = h
        ee = lax.dot_general(x, q, (((1,), (1,)), ((), ())),
                             preferred_element_type=jnp.float32)
        e = jnp.sum(ee * mtf, axis=1, keepdims=True)
        emax = jnp.max(jnp.where(mtb, jnp.broadcast_to(e, (NP, 16)), -jnp.inf),
                       axis=0, keepdims=True)
        emax = jnp.where(emax > -1e30, emax, 0.0)
        emax_sel = jnp.sum(emax * mtf, axis=1, keepdims=True)
        ex = jnp.exp(e - emax_sel)
        denom = jnp.sum(mtf * jnp.broadcast_to(ex, (NP, 16)), axis=0, keepdims=True)
        denom_sel = jnp.sum(denom * mtf, axis=1, keepdims=True)
        a = ex / (denom_sel + 1e-16)
        r = jnp.dot(mf, a * x, preferred_element_type=jnp.float32)
        q_star = jnp.concatenate([q, r], axis=1)
    return q_star


def _tc_final_body(h_ref, b1c_ref, b1r_ref, b2c_ref, b2r_ref,
                   wih1_ref, whh1_ref, bih1_ref, bhh1_ref,
                   wih2_ref, whh2_ref, bih2_ref, bhh2_ref,
                   w1_ref, b1_ref, w2_ref, b2_ref, o_ref):
    g1 = _set2set_in_kernel(h_ref[0:NP], b1c_ref[...], b1r_ref[...],
                            wih1_ref[...], whh1_ref[...], bih1_ref[...], bhh1_ref[...])
    g2 = _set2set_in_kernel(h_ref[NP:NP2], b2c_ref[...], b2r_ref[...],
                            wih2_ref[...], whh2_ref[...], bih2_ref[...], bhh2_ref[...])
    cat = jnp.concatenate([g1, g2], axis=1)
    hfc = jnp.maximum(
        lax.dot_general(cat, w1_ref[...], (((1,), (1,)), ((), ())),
                        preferred_element_type=jnp.float32) + b1_ref[...], 0.0)
    o_ref[...] = jnp.sum(hfc * w2_ref[...], axis=1, keepdims=True) + b2_ref[...]


def _tc_final(hcat, b1c, b1r, b2c, b2r, p1, p2, fc1_W, fc1_b, fc2_W, fc2_b):
    return pl.pallas_call(
        _tc_final_body,
        out_shape=jax.ShapeDtypeStruct((B, 1), jnp.float32),
    )(hcat, b1c, b1r, b2c, b2r,
      p1[0], p1[1], p1[2], p1[3],
      p2[0], p2[1], p2[2], p2[3],
      fc1_W, fc1_b[None, :], fc2_W, fc2_b[None, :])


def kernel(x1, x2, edge_index1, edge_index2, x1_batch, x2_batch,
           e1_lin0_W, e1_lin0_b, e1_conv_W, e1_conv_b,
           e1_gru_Wih, e1_gru_Whh, e1_gru_bih, e1_gru_bhh,
           e1_lstm_Wih, e1_lstm_Whh, e1_lstm_bih, e1_lstm_bhh,
           e2_lin0_W, e2_lin0_b, e2_conv_W, e2_conv_b,
           e2_gru_Wih, e2_gru_Whh, e2_gru_bih, e2_gru_bhh,
           e2_lstm_Wih, e2_lstm_Whh, e2_lstm_bih, e2_lstm_bhh,
           fc1_W, fc1_b, fc2_W, fc2_b):
    f32 = jnp.float32
    epad = ECH * CH - E

    def edges2d(v, off):
        return jnp.pad(v, (0, epad), constant_values=N).reshape(ECH, CH) + off

    src1 = edges2d(edge_index1[0], 0)
    dst1 = edges2d(edge_index1[1], 0)
    src2 = edges2d(edge_index2[0], NP)
    dst2 = edges2d(edge_index2[1], 0)
    zbig = jnp.zeros((NP, D), f32)
    ones_h = jnp.ones((CH, D), f32)

    degcat = _sc_deg(dst1, dst2, zbig, ones_h)[:, 0:16]

    xcat = jnp.concatenate([
        jnp.pad(x1, ((0, NP - N), (0, 0))),
        jnp.pad(x2, ((0, NP - N), (0, 0))),
    ], axis=0)
    w0 = jnp.stack([e1_lin0_W, e2_lin0_W])
    b0 = jnp.stack([e1_lin0_b[None, :], e2_lin0_b[None, :]])
    wc = jnp.stack([e1_conv_W, e2_conv_W])
    bc = jnp.stack([e1_conv_b[None, :], e2_conv_b[None, :]])
    wih = jnp.stack([e1_gru_Wih, e2_gru_Wih])
    whh = jnp.stack([e1_gru_Whh, e2_gru_Whh])
    bih = jnp.stack([e1_gru_bih[None, :], e2_gru_bih[None, :]])
    bhh = jnp.stack([e1_gru_bhh[None, :], e2_gru_bhh[None, :]])

    hcat, ycat = _tc_prep(xcat, degcat, w0, b0, wc)
    hcat, ycat = hcat[0], ycat[0]
    for _ in range(3):
        acccat = _sc_agg(ycat, src1, dst1, src2, dst2, zbig)
        hcat, ycat = _tc_layer(acccat, ycat, hcat, degcat, wc, wih, whh, bih, bhh, bc)
        hcat, ycat = hcat[0], ycat[0]

    bpad = NP - N
    b1p = jnp.pad(x1_batch, (0, bpad), constant_values=B)
    b2p = jnp.pad(x2_batch, (0, bpad), constant_values=B)
    b1c = jnp.broadcast_to(b1p[:, None], (NP, 16))
    b2c = jnp.broadcast_to(b2p[:, None], (NP, 16))
    b1r = jnp.broadcast_to(b1p[None, :], (B, NP))
    b2r = jnp.broadcast_to(b2p[None, :], (B, NP))
    p1 = (e1_lstm_Wih, e1_lstm_Whh, e1_lstm_bih[None, :], e1_lstm_bhh[None, :])
    p2 = (e2_lstm_Wih, e2_lstm_Whh, e2_lstm_bih[None, :], e2_lstm_bhh[None, :])
    out = _tc_final(hcat, b1c, b1r, b2c, b2r, p1, p2, fc1_W, fc1_b, fc2_W, fc2_b)
    return out.reshape(-1)


# speedup vs baseline: 1.1681x; 1.0253x over previous
import functools

import jax
import jax.numpy as jnp
from jax import lax
from jax.experimental import pallas as pl
from jax.experimental.pallas import tpu as pltpu
from jax.experimental.pallas import tpu_sc as plsc

N = 10000
E = 320000
D = 128
B = 16
NP = 10240
NP2 = 2 * NP
NSC = 2
NTL = 16
CH = 128
ECH = 2560
TCH = ECH // NTL
RPT = NP // NTL
BM = 1024
NBLK = NP2 // BM


def _mesh():
    return plsc.VectorSubcoreMesh(
        core_axis_name="c", subcore_axis_name="s", num_cores=NSC, num_subcores=NTL
    )


def _sc_deg_body(dst1, dst2, zbig, ones_h, out, idx_v, ones_v, acc, sem):
    c = lax.axis_index("c")
    s = lax.axis_index("s")
    pltpu.sync_copy(ones_h, ones_v)
    pltpu.sync_copy(zbig.at[pl.ds(s * RPT, RPT)], acc.at[pl.ds(s * RPT, RPT)])

    @pl.when(c == 0)
    def _():
        pltpu.sync_copy(dst1.at[pl.ds(s * TCH, TCH)], idx_v)

    @pl.when(c == 1)
    def _():
        pltpu.sync_copy(dst2.at[pl.ds(s * TCH, TCH)], idx_v)

    plsc.subcore_barrier()

    def body(j, carry):
        pltpu.sync_copy(ones_v, acc.at[idx_v.at[j]], add=True)
        return carry

    lax.fori_loop(0, TCH, body, 0)
    plsc.subcore_barrier()
    pltpu.sync_copy(acc.at[pl.ds(s * RPT, RPT)], out.at[pl.ds(c * NP + s * RPT, RPT)])


def _sc_deg(dst1, dst2, zbig, ones_h):
    return pl.kernel(
        _sc_deg_body,
        out_type=jax.ShapeDtypeStruct((NP2, D), jnp.float32),
        mesh=_mesh(),
        scratch_types=[
            pltpu.VMEM((TCH, CH), jnp.int32),
            pltpu.VMEM((CH, D), jnp.float32),
            pltpu.VMEM_SHARED((NP, D), jnp.float32),
            pltpu.SemaphoreType.DMA,
        ],
    )(dst1, dst2, zbig, ones_h)


STG = 40
NSTG = TCH // STG
UNR = 20


def _sc_agg_body(ycat, src1, dst1, src2, dst2, zbig, out,
                 src_v, dst_v, rows, rows1, acc, sem, sem1):
    c = lax.axis_index("c")
    s = lax.axis_index("s")
    pltpu.sync_copy(zbig.at[pl.ds(s * RPT, RPT)], acc.at[pl.ds(s * RPT, RPT)])
    plsc.subcore_barrier()

    def run(src_hbm, dst_hbm):
        bufs = (rows, rows1)
        sems = (sem, sem1)

        def stage(st, carry):
            pltpu.sync_copy(src_hbm.at[pl.ds(s * TCH + st * STG, STG)], src_v)
            pltpu.sync_copy(dst_hbm.at[pl.ds(s * TCH + st * STG, STG)], dst_v)

            def block(ko, carry2):
                base = ko * UNR
                ds = [
                    pltpu.async_copy(ycat.at[src_v.at[base]], bufs[0], sems[0]),
                    pltpu.async_copy(ycat.at[src_v.at[base + 1]], bufs[1], sems[1]),
                ]
                for k in range(UNR):
                    ds[k].wait()
                    pltpu.sync_copy(bufs[k % 2], acc.at[dst_v.at[base + k]], add=True)
                    if k + 2 < UNR:
                        ds.append(pltpu.async_copy(
                            ycat.at[src_v.at[base + k + 2]], bufs[k % 2], sems[k % 2]))
                return carry2

            return lax.fori_loop(0, STG // UNR, block, carry)

        lax.fori_loop(0, NSTG, stage, 0)

    @pl.when(c == 0)
    def _():
        run(src1, dst1)

    @pl.when(c == 1)
    def _():
        run(src2, dst2)

    plsc.subcore_barrier()
    pltpu.sync_copy(acc.at[pl.ds(s * RPT, RPT)], out.at[pl.ds(c * NP + s * RPT, RPT)])


def _sc_agg(ycat, src1, dst1, src2, dst2, zbig):
    return pl.kernel(
        _sc_agg_body,
        out_type=jax.ShapeDtypeStruct((NP2, D), jnp.float32),
        mesh=_mesh(),
        scratch_types=[
            pltpu.VMEM((STG, CH), jnp.int32),
            pltpu.VMEM((STG, CH), jnp.int32),
            pltpu.VMEM((CH, D), jnp.float32),
            pltpu.VMEM((CH, D), jnp.float32),
            pltpu.VMEM_SHARED((NP, D), jnp.float32),
            pltpu.SemaphoreType.DMA,
            pltpu.SemaphoreType.DMA,
        ],
    )(ycat, src1, dst1, src2, dst2, zbig)


def _tc_prep_body(x_ref, deg_ref, w0_ref, b0_ref, wc_ref, out_ref, y_ref):
    x = x_ref[0]
    o = jnp.maximum(
        lax.dot_general(x, w0_ref[0], (((1,), (1,)), ((), ())),
                        preferred_element_type=jnp.float32) + b0_ref[0], 0.0)
    dinv = lax.rsqrt(deg_ref[0][:, 0:1] + 1.0)
    out_ref[0] = o
    y_ref[0] = dinv * jnp.dot(o, wc_ref[0], preferred_element_type=jnp.float32)


def _tc_prep(xcat, degcat, w0, b0, wc):
    f32 = jnp.float32
    return pl.pallas_call(
        _tc_prep_body,
        grid=(NBLK,),
        in_specs=[
            pl.BlockSpec((1, BM, D), lambda i: (0, i, 0)),
            pl.BlockSpec((1, BM, 16), lambda i: (0, i, 0)),
            pl.BlockSpec((1, D, D), lambda i: (i // (NBLK // 2), 0, 0)),
            pl.BlockSpec((1, 1, D), lambda i: (i // (NBLK // 2), 0, 0)),
            pl.BlockSpec((1, D, D), lambda i: (i // (NBLK // 2), 0, 0)),
        ],
        out_specs=[
            pl.BlockSpec((1, BM, D), lambda i: (0, i, 0)),
            pl.BlockSpec((1, BM, D), lambda i: (0, i, 0)),
        ],
        out_shape=[
            jax.ShapeDtypeStruct((1, NP2, D), f32),
            jax.ShapeDtypeStruct((1, NP2, D), f32),
        ],
    )(xcat[None], degcat[None], w0, b0, wc)


def _tc_layer_body(acc_ref, y_ref, h_ref, deg_ref, wc_ref, wih_ref, whh_ref,
                   bih_ref, bhh_ref, bc_ref, hn_ref, yn_ref):
    dinv = lax.rsqrt(deg_ref[0][:, 0:1] + 1.0)
    m = jnp.maximum(dinv * (acc_ref[0] + y_ref[0]) + bc_ref[0], 0.0)
    h = h_ref[0]
    gi = lax.dot_general(m, wih_ref[0], (((1,), (1,)), ((), ())),
                         preferred_element_type=jnp.float32) + bih_ref[0]
    gh = lax.dot_general(h, whh_ref[0], (((1,), (1,)), ((), ())),
                         preferred_element_type=jnp.float32) + bhh_ref[0]
    r = jax.nn.sigmoid(gi[:, 0:D] + gh[:, 0:D])
    z = jax.nn.sigmoid(gi[:, D:2 * D] + gh[:, D:2 * D])
    nn_ = jnp.tanh(gi[:, 2 * D:3 * D] + r * gh[:, 2 * D:3 * D])
    hn = (1.0 - z) * nn_ + z * h
    hn_ref[0] = hn
    yn_ref[0] = dinv * jnp.dot(hn, wc_ref[0], preferred_element_type=jnp.float32)


def _tc_layer(acccat, ycat, hcat, degcat, wc, wih, whh, bih, bhh, bc):
    f32 = jnp.float32
    wspec = lambda r, c: pl.BlockSpec((1, r, c), lambda i: (i // (NBLK // 2), 0, 0))
    return pl.pallas_call(
        _tc_layer_body,
        grid=(NBLK,),
        in_specs=[
            pl.BlockSpec((1, BM, D), lambda i: (0, i, 0)),
            pl.BlockSpec((1, BM, D), lambda i: (0, i, 0)),
            pl.BlockSpec((1, BM, D), lambda i: (0, i, 0)),
            pl.BlockSpec((1, BM, 16), lambda i: (0, i, 0)),
            wspec(D, D),
            wspec(3 * D, D),
            wspec(3 * D, D),
            wspec(1, 3 * D),
            wspec(1, 3 * D),
            wspec(1, D),
        ],
        out_specs=[
            pl.BlockSpec((1, BM, D), lambda i: (0, i, 0)),
            pl.BlockSpec((1, BM, D), lambda i: (0, i, 0)),
        ],
        out_shape=[
            jax.ShapeDtypeStruct((1, NP2, D), f32),
            jax.ShapeDtypeStruct((1, NP2, D), f32),
        ],
    )(acccat[None], ycat[None], hcat[None], degcat[None], wc, wih, whh, bih, bhh, bc)


def _set2set_in_kernel(x, bcol16, brow16, wih, whh, bih, bhh):
    mtb = bcol16 == lax.broadcasted_iota(jnp.int32, (NP, 16), 1)
    mtf = mtb.astype(jnp.float32)
    mf = (brow16 == lax.broadcasted_iota(jnp.int32, (B, NP), 0)).astype(jnp.float32)
    h = jnp.zeros((B, D), jnp.float32)
    cc = jnp.zeros((B, D), jnp.float32)
    q_star = jnp.zeros((B, 2 * D), jnp.float32)
    for _ in range(3):
        gates = (lax.dot_general(q_star, wih, (((1,), (1,)), ((), ())),
                                 preferred_element_type=jnp.float32) + bih
                 + lax.dot_general(h, whh, (((1,), (1,)), ((), ())),
                                   preferred_element_type=jnp.float32) + bhh)
        ii = jax.nn.sigmoid(gates[:, 0:D])
        ff = jax.nn.sigmoid(gates[:, D:2 * D])
        gg = jnp.tanh(gates[:, 2 * D:3 * D])
        oo = jax.nn.sigmoid(gates[:, 3 * D:4 * D])
        cc = ff * cc + ii * gg
        h = oo * jnp.tanh(cc)
        q = h
        ee = lax.dot_general(x, q, (((1,), (1,)), ((), ())),
                             preferred_element_type=jnp.float32)
        e = jnp.sum(ee * mtf, axis=1, keepdims=True)
        emax = jnp.max(jnp.where(mtb, jnp.broadcast_to(e, (NP, 16)), -jnp.inf),
                       axis=0, keepdims=True)
        emax = jnp.where(emax > -1e30, emax, 0.0)
        emax_sel = jnp.sum(emax * mtf, axis=1, keepdims=True)
        ex = jnp.exp(e - emax_sel)
        denom = jnp.sum(mtf * jnp.broadcast_to(ex, (NP, 16)), axis=0, keepdims=True)
        denom_sel = jnp.sum(denom * mtf, axis=1, keepdims=True)
        a = ex / (denom_sel + 1e-16)
        r = jnp.dot(mf, a * x, preferred_element_type=jnp.float32)
        q_star = jnp.concatenate([q, r], axis=1)
    return q_star


def _tc_final_body(h_ref, b1c_ref, b1r_ref, b2c_ref, b2r_ref,
                   wih1_ref, whh1_ref, bih1_ref, bhh1_ref,
                   wih2_ref, whh2_ref, bih2_ref, bhh2_ref,
                   w1_ref, b1_ref, w2_ref, b2_ref, o_ref):
    g1 = _set2set_in_kernel(h_ref[0:NP], b1c_ref[...], b1r_ref[...],
                            wih1_ref[...], whh1_ref[...], bih1_ref[...], bhh1_ref[...])
    g2 = _set2set_in_kernel(h_ref[NP:NP2], b2c_ref[...], b2r_ref[...],
                            wih2_ref[...], whh2_ref[...], bih2_ref[...], bhh2_ref[...])
    cat = jnp.concatenate([g1, g2], axis=1)
    hfc = jnp.maximum(
        lax.dot_general(cat, w1_ref[...], (((1,), (1,)), ((), ())),
                        preferred_element_type=jnp.float32) + b1_ref[...], 0.0)
    o_ref[...] = jnp.sum(hfc * w2_ref[...], axis=1, keepdims=True) + b2_ref[...]


def _tc_final(hcat, b1c, b1r, b2c, b2r, p1, p2, fc1_W, fc1_b, fc2_W, fc2_b):
    return pl.pallas_call(
        _tc_final_body,
        out_shape=jax.ShapeDtypeStruct((B, 1), jnp.float32),
    )(hcat, b1c, b1r, b2c, b2r,
      p1[0], p1[1], p1[2], p1[3],
      p2[0], p2[1], p2[2], p2[3],
      fc1_W, fc1_b[None, :], fc2_W, fc2_b[None, :])


def kernel(x1, x2, edge_index1, edge_index2, x1_batch, x2_batch,
           e1_lin0_W, e1_lin0_b, e1_conv_W, e1_conv_b,
           e1_gru_Wih, e1_gru_Whh, e1_gru_bih, e1_gru_bhh,
           e1_lstm_Wih, e1_lstm_Whh, e1_lstm_bih, e1_lstm_bhh,
           e2_lin0_W, e2_lin0_b, e2_conv_W, e2_conv_b,
           e2_gru_Wih, e2_gru_Whh, e2_gru_bih, e2_gru_bhh,
           e2_lstm_Wih, e2_lstm_Whh, e2_lstm_bih, e2_lstm_bhh,
           fc1_W, fc1_b, fc2_W, fc2_b):
    f32 = jnp.float32
    epad = ECH * CH - E

    def edges2d(v, off):
        return jnp.pad(v, (0, epad), constant_values=N).reshape(ECH, CH) + off

    src1 = edges2d(edge_index1[0], 0)
    dst1 = edges2d(edge_index1[1], 0)
    src2 = edges2d(edge_index2[0], NP)
    dst2 = edges2d(edge_index2[1], 0)
    zbig = jnp.zeros((NP, D), f32)
    ones_h = jnp.ones((CH, D), f32)

    degcat = _sc_deg(dst1, dst2, zbig, ones_h)[:, 0:16]

    xcat = jnp.concatenate([
        jnp.pad(x1, ((0, NP - N), (0, 0))),
        jnp.pad(x2, ((0, NP - N), (0, 0))),
    ], axis=0)
    w0 = jnp.stack([e1_lin0_W, e2_lin0_W])
    b0 = jnp.stack([e1_lin0_b[None, :], e2_lin0_b[None, :]])
    wc = jnp.stack([e1_conv_W, e2_conv_W])
    bc = jnp.stack([e1_conv_b[None, :], e2_conv_b[None, :]])
    wih = jnp.stack([e1_gru_Wih, e2_gru_Wih])
    whh = jnp.stack([e1_gru_Whh, e2_gru_Whh])
    bih = jnp.stack([e1_gru_bih[None, :], e2_gru_bih[None, :]])
    bhh = jnp.stack([e1_gru_bhh[None, :], e2_gru_bhh[None, :]])

    hcat, ycat = _tc_prep(xcat, degcat, w0, b0, wc)
    hcat, ycat = hcat[0], ycat[0]
    for _ in range(3):
        acccat = _sc_agg(ycat, src1, dst1, src2, dst2, zbig)
        hcat, ycat = _tc_layer(acccat, ycat, hcat, degcat, wc, wih, whh, bih, bhh, bc)
        hcat, ycat = hcat[0], ycat[0]

    bpad = NP - N
    b1p = jnp.pad(x1_batch, (0, bpad), constant_values=B)
    b2p = jnp.pad(x2_batch, (0, bpad), constant_values=B)
    b1c = jnp.broadcast_to(b1p[:, None], (NP, 16))
    b2c = jnp.broadcast_to(b2p[:, None], (NP, 16))
    b1r = jnp.broadcast_to(b1p[None, :], (B, NP))
    b2r = jnp.broadcast_to(b2p[None, :], (B, NP))
    p1 = (e1_lstm_Wih, e1_lstm_Whh, e1_lstm_bih[None, :], e1_lstm_bhh[None, :])
    p2 = (e2_lstm_Wih, e2_lstm_Whh, e2_lstm_bih[None, :], e2_lstm_bhh[None, :])
    out = _tc_final(hcat, b1c, b1r, b2c, b2r, p1, p2, fc1_W, fc1_b, fc2_W, fc2_b)
    return out.reshape(-1)
